# initial kernel scaffold (unmeasured)
import jax
import jax.numpy as jnp
from jax import lax
from jax.experimental import pallas as pl
from jax.experimental.pallas import tpu as pltpu

N_DEV = 4
NGROUP = 4


def kernel(O, Wo):
    B, S, Hs, D = O.shape
    K = Hs * D
    N = Wo.shape[1]
    S_out = S // N_DEV
    W = N // NGROUP

    partial = lax.dot_general(
        O.reshape(B, S, K).astype(jnp.bfloat16),
        Wo.astype(jnp.bfloat16),
        (((2,), (0,)), ((), ())),
        preferred_element_type=jnp.bfloat16,
    )

    def body(p_ref, o_ref, send_buf, recv_buf, local_buf, out_stage,
             send_sem, recv_sem, local_sem, out_sem, credit_sem):
        i = lax.axis_index("i")
        left = (i + N_DEV - 1) % N_DEV
        right = (i + 1) % N_DEV

        barrier = pltpu.get_barrier_semaphore()
        for nbr in (left, right):
            pl.semaphore_signal(
                barrier, inc=1,
                device_id=(nbr,), device_id_type=pl.DeviceIdType.MESH,
            )
        pl.semaphore_wait(barrier, 2)

        n_hops = NGROUP * (N_DEV - 1)
        k = 0
        for g in range(NGROUP):
            col = g * W
            c0 = (i + N_DEV - 1) % N_DEV
            cp = pltpu.make_async_copy(
                p_ref.at[:, pl.ds(c0 * S_out, S_out), pl.ds(col, W)],
                send_buf, local_sem)
            cp.start()
            cp.wait()

            for t in range(N_DEV - 1):
                if k > 0:
                    pl.semaphore_wait(credit_sem, 1)
                rdma = pltpu.make_async_remote_copy(
                    src_ref=send_buf, dst_ref=recv_buf,
                    send_sem=send_sem, recv_sem=recv_sem,
                    device_id=(right,), device_id_type=pl.DeviceIdType.MESH,
                )
                rdma.start()

                c = (i + 2 * N_DEV - 2 - t) % N_DEV
                cp = pltpu.make_async_copy(
                    p_ref.at[:, pl.ds(c * S_out, S_out), pl.ds(col, W)],
                    local_buf, local_sem)
                cp.start()
                cp.wait()

                rdma.wait_recv()
                rdma.wait_send()
                if t < N_DEV - 2:
                    send_buf[...] = recv_buf[...] + local_buf[...]
                else:
                    out_stage[...] = (recv_buf[...].astype(jnp.float32)
                                      + local_buf[...].astype(jnp.float32))
                if k < n_hops - 1:
                    pl.semaphore_signal(
                        credit_sem, inc=1,
                        device_id=(left,), device_id_type=pl.DeviceIdType.MESH,
                    )
                if t == N_DEV - 2:
                    ocp = pltpu.make_async_copy(
                        out_stage, o_ref.at[:, :, pl.ds(col, W)], out_sem)
                    ocp.start()
                    ocp.wait()
                k += 1

    return pl.pallas_call(
        body,
        out_shape=jax.ShapeDtypeStruct((B, S_out, N), jnp.float32),
        in_specs=[pl.BlockSpec(memory_space=pltpu.ANY)],
        out_specs=pl.BlockSpec(memory_space=pltpu.ANY),
        scratch_shapes=[
            pltpu.VMEM((B, S_out, W), jnp.bfloat16),
            pltpu.VMEM((B, S_out, W), jnp.bfloat16),
            pltpu.VMEM((B, S_out, W), jnp.bfloat16),
            pltpu.VMEM((B, S_out, W), jnp.float32),
            pltpu.SemaphoreType.DMA,
            pltpu.SemaphoreType.DMA,
            pltpu.SemaphoreType.DMA,
            pltpu.SemaphoreType.DMA,
            pltpu.SemaphoreType.REGULAR,
        ],
        compiler_params=pltpu.CompilerParams(collective_id=0),
    )(partial)


# baseline (device time: 1614960 ns/iter reference)
import jax
import jax.numpy as jnp
from jax import lax
from jax.experimental import pallas as pl
from jax.experimental.pallas import tpu as pltpu

N_DEV = 4
NGROUP = 4


def kernel(O, Wo):
    B, S, Hs, D = O.shape
    K = Hs * D
    N = Wo.shape[1]
    S_out = S // N_DEV
    W = N // NGROUP

    partial = lax.dot_general(
        O.reshape(B, S, K).astype(jnp.bfloat16),
        Wo.astype(jnp.bfloat16),
        (((2,), (0,)), ((), ())),
        preferred_element_type=jnp.bfloat16,
    )

    def body(p_ref, o_ref, send_buf, recv_buf, local_buf, out_stage,
             send_sem, recv_sem, local_sem, out_sem, credit_sem):
        i = lax.axis_index("i")
        left = (i + N_DEV - 1) % N_DEV
        right = (i + 1) % N_DEV

        barrier = pltpu.get_barrier_semaphore()
        for nbr in (left, right):
            pl.semaphore_signal(
                barrier, inc=1,
                device_id=(nbr,), device_id_type=pl.DeviceIdType.MESH,
            )
        pl.semaphore_wait(barrier, 2)

        n_hops = NGROUP * (N_DEV - 1)
        k = 0
        for g in range(NGROUP):
            col = g * W
            c0 = (i + N_DEV - 1) % N_DEV
            cp = pltpu.make_async_copy(
                p_ref.at[:, pl.ds(c0 * S_out, S_out), pl.ds(col, W)],
                send_buf, local_sem)
            cp.start()
            cp.wait()

            for t in range(N_DEV - 1):
                if k > 0:
                    pl.semaphore_wait(credit_sem, 1)
                rdma = pltpu.make_async_remote_copy(
                    src_ref=send_buf, dst_ref=recv_buf,
                    send_sem=send_sem, recv_sem=recv_sem,
                    device_id=(right,), device_id_type=pl.DeviceIdType.MESH,
                )
                rdma.start()

                c = (i + 2 * N_DEV - 2 - t) % N_DEV
                cp = pltpu.make_async_copy(
                    p_ref.at[:, pl.ds(c * S_out, S_out), pl.ds(col, W)],
                    local_buf, local_sem)
                cp.start()
                cp.wait()

                rdma.wait_recv()
                rdma.wait_send()
                if t < N_DEV - 2:
                    send_buf[...] = recv_buf[...] + local_buf[...]
                else:
                    out_stage[...] = (recv_buf[...].astype(jnp.float32)
                                      + local_buf[...].astype(jnp.float32))
                if k < n_hops - 1:
                    pl.semaphore_signal(
                        credit_sem, inc=1,
                        device_id=(left,), device_id_type=pl.DeviceIdType.MESH,
                    )
                if t == N_DEV - 2:
                    ocp = pltpu.make_async_copy(
                        out_stage, o_ref.at[:, :, pl.ds(col, W)], out_sem)
                    ocp.start()
                    ocp.wait()
                k += 1

    return pl.pallas_call(
        body,
        out_shape=jax.ShapeDtypeStruct((B, S_out, N), jnp.float32),
        in_specs=[pl.BlockSpec(memory_space=pl.ANY)],
        out_specs=pl.BlockSpec(memory_space=pl.ANY),
        scratch_shapes=[
            pltpu.VMEM((B, S_out, W), jnp.bfloat16),
            pltpu.VMEM((B, S_out, W), jnp.bfloat16),
            pltpu.VMEM((B, S_out, W), jnp.bfloat16),
            pltpu.VMEM((B, S_out, W), jnp.float32),
            pltpu.SemaphoreType.DMA,
            pltpu.SemaphoreType.DMA,
            pltpu.SemaphoreType.DMA,
            pltpu.SemaphoreType.DMA,
            pltpu.SemaphoreType.REGULAR,
        ],
        compiler_params=pltpu.CompilerParams(
            collective_id=0,
            vmem_limit_bytes=56 * 1024 * 1024,
        ),
    )(partial)


# device time: 1088556 ns/iter; 1.4836x vs baseline; 1.4836x over previous
import jax
import jax.numpy as jnp
from jax import lax
from jax.experimental import pallas as pl
from jax.experimental.pallas import tpu as pltpu

N_DEV = 4
NGROUP = 4
N_STEP = (NGROUP // 2) * (N_DEV - 1)


def kernel(O, Wo):
    B, S, Hs, D = O.shape
    K = Hs * D
    N = Wo.shape[1]
    S_out = S // N_DEV
    W = N // NGROUP

    partial = lax.dot_general(
        O.reshape(B, S, K).astype(jnp.bfloat16),
        Wo.astype(jnp.bfloat16),
        (((2,), (0,)), ((), ())),
        preferred_element_type=jnp.bfloat16,
    )

    def body(p_ref, o_ref, send_cw, recv_cw, send_ccw, recv_ccw, out_stage,
             send_sem_cw, recv_sem_cw, send_sem_ccw, recv_sem_ccw,
             copy_sem_cw, copy_sem_ccw, out_sem, credit_cw, credit_ccw):
        i = lax.axis_index("i")
        left = (i + N_DEV - 1) % N_DEV
        right = (i + 1) % N_DEV

        barrier = pltpu.get_barrier_semaphore()
        for nbr in (left, right):
            pl.semaphore_signal(
                barrier, inc=1,
                device_id=(nbr,), device_id_type=pl.DeviceIdType.MESH,
            )
        pl.semaphore_wait(barrier, 2)

        def load(dst, chunk, col, sem):
            cp = pltpu.make_async_copy(
                p_ref.at[:, pl.ds(chunk * S_out, S_out), pl.ds(col, W)],
                dst, sem)
            cp.start()
            return cp

        k = 0
        for gg in range(NGROUP // 2):
            col_cw = gg * W
            col_ccw = (NGROUP // 2 + gg) * W
            cp1 = load(send_cw, (i + N_DEV - 1) % N_DEV, col_cw, copy_sem_cw)
            cp2 = load(send_ccw, (i + 1) % N_DEV, col_ccw, copy_sem_ccw)
            cp1.wait()
            cp2.wait()

            for t in range(N_DEV - 1):
                if k > 0:
                    pl.semaphore_wait(credit_cw, 1)
                    pl.semaphore_wait(credit_ccw, 1)
                rdma_cw = pltpu.make_async_remote_copy(
                    src_ref=send_cw, dst_ref=recv_cw,
                    send_sem=send_sem_cw, recv_sem=recv_sem_cw,
                    device_id=(right,), device_id_type=pl.DeviceIdType.MESH,
                )
                rdma_ccw = pltpu.make_async_remote_copy(
                    src_ref=send_ccw, dst_ref=recv_ccw,
                    send_sem=send_sem_ccw, recv_sem=recv_sem_ccw,
                    device_id=(left,), device_id_type=pl.DeviceIdType.MESH,
                )
                rdma_cw.start()
                rdma_ccw.start()

                c_cw = (i + 2 * N_DEV - 2 - t) % N_DEV
                c_ccw = (i + 2 + t) % N_DEV
                rdma_cw.wait_send()
                cp1 = load(send_cw, c_cw, col_cw, copy_sem_cw)
                rdma_ccw.wait_send()
                cp2 = load(send_ccw, c_ccw, col_ccw, copy_sem_ccw)
                cp1.wait()
                cp2.wait()

                last = t == N_DEV - 2
                rdma_cw.wait_recv()
                if not last:
                    send_cw[...] = send_cw[...] + recv_cw[...]
                else:
                    out_stage[...] = (send_cw[...].astype(jnp.float32)
                                      + recv_cw[...].astype(jnp.float32))
                if k < N_STEP - 1:
                    pl.semaphore_signal(
                        credit_cw, inc=1,
                        device_id=(left,), device_id_type=pl.DeviceIdType.MESH,
                    )
                if last:
                    ocp = pltpu.make_async_copy(
                        out_stage, o_ref.at[:, :, pl.ds(col_cw, W)], out_sem)
                    ocp.start()
                    ocp.wait()

                rdma_ccw.wait_recv()
                if not last:
                    send_ccw[...] = send_ccw[...] + recv_ccw[...]
                else:
                    out_stage[...] = (send_ccw[...].astype(jnp.float32)
                                      + recv_ccw[...].astype(jnp.float32))
                if k < N_STEP - 1:
                    pl.semaphore_signal(
                        credit_ccw, inc=1,
                        device_id=(right,), device_id_type=pl.DeviceIdType.MESH,
                    )
                if last:
                    ocp = pltpu.make_async_copy(
                        out_stage, o_ref.at[:, :, pl.ds(col_ccw, W)], out_sem)
                    ocp.start()
                    ocp.wait()
                k += 1

    return pl.pallas_call(
        body,
        out_shape=jax.ShapeDtypeStruct((B, S_out, N), jnp.float32),
        in_specs=[pl.BlockSpec(memory_space=pl.ANY)],
        out_specs=pl.BlockSpec(memory_space=pl.ANY),
        scratch_shapes=[
            pltpu.VMEM((B, S_out, W), jnp.bfloat16),
            pltpu.VMEM((B, S_out, W), jnp.bfloat16),
            pltpu.VMEM((B, S_out, W), jnp.bfloat16),
            pltpu.VMEM((B, S_out, W), jnp.bfloat16),
            pltpu.VMEM((B, S_out, W), jnp.float32),
            pltpu.SemaphoreType.DMA,
            pltpu.SemaphoreType.DMA,
            pltpu.SemaphoreType.DMA,
            pltpu.SemaphoreType.DMA,
            pltpu.SemaphoreType.DMA,
            pltpu.SemaphoreType.DMA,
            pltpu.SemaphoreType.DMA,
            pltpu.SemaphoreType.REGULAR,
            pltpu.SemaphoreType.REGULAR,
        ],
        compiler_params=pltpu.CompilerParams(
            collective_id=0,
            vmem_limit_bytes=56 * 1024 * 1024,
        ),
    )(partial)


# device time: 880478 ns/iter; 1.8342x vs baseline; 1.2363x over previous
import os as _os
import jax
from pathlib import Path as _Path

_CACHE_DIR = str(_Path(__file__).parent / ".jax_cache")
_os.makedirs(_CACHE_DIR, exist_ok=True)
for _k, _v in [
    ("jax_compilation_cache_dir", _CACHE_DIR),
    ("jax_persistent_cache_min_entry_size_bytes", 0),
    ("jax_persistent_cache_min_compile_time_secs", 0.0),
]:
    try:
        jax.config.update(_k, _v)
    except Exception:
        pass

import jax.numpy as jnp
from jax import lax
from jax.experimental import pallas as pl
from jax.experimental.pallas import tpu as pltpu

N_DEV = 4
NGROUP = 8
NG_DIR = NGROUP // 2
N_STEP = NG_DIR * (N_DEV - 1)


def kernel(O, Wo):
    B, S, Hs, D = O.shape
    K = Hs * D
    N = Wo.shape[1]
    S_out = S // N_DEV
    W = N // NGROUP
    f32 = jnp.float32
    bf16 = jnp.bfloat16

    Ob = O.reshape(B, S, K).astype(bf16)
    Wob = Wo.astype(bf16)

    def body(o_in, w_in, o_ref,
             lhs, pan_cw, pan_ccw, acc_cw, acc_ccw,
             send_cw, recv_cw, send_ccw, recv_ccw,
             send_sem_cw, recv_sem_cw, send_sem_ccw, recv_sem_ccw,
             lhs_sem, pan_sem, out_sem,
             credit_cw, credit_ccw):
        i = lax.axis_index("i")
        left = (i + N_DEV - 1) % N_DEV
        right = (i + 1) % N_DEV

        barrier = pltpu.get_barrier_semaphore()
        for nbr in (left, right):
            pl.semaphore_signal(
                barrier, inc=1,
                device_id=(nbr,), device_id_type=pl.DeviceIdType.MESH,
            )
        pl.semaphore_wait(barrier, 2)

        def load_lhs(dst, chunk, sem):
            cp = pltpu.make_async_copy(
                o_in.at[:, pl.ds(chunk * S_out, S_out), :], dst, sem)
            cp.start()
            return cp

        def load_pan(dst, col, sem):
            cp = pltpu.make_async_copy(w_in.at[:, pl.ds(col, W)], dst, sem)
            cp.start()
            return cp

        def mm(lhs_ref, pan_ref, acc_ref):
            x = lhs_ref[...].reshape(B * S_out, K)
            acc_ref[...] = jnp.dot(
                x, pan_ref[...], preferred_element_type=f32,
            ).reshape(B, S_out, W)

        k = 0
        for gg in range(NG_DIR):
            col_cw = gg * W
            col_ccw = (NG_DIR + gg) * W
            p1 = load_pan(pan_cw, col_cw, pan_sem)
            p2 = load_pan(pan_ccw, col_ccw, out_sem)
            l1 = load_lhs(lhs, (i + N_DEV - 1) % N_DEV, lhs_sem)
            p1.wait()
            p2.wait()
            l1.wait()
            mm(lhs, pan_cw, acc_cw)
            send_cw[...] = acc_cw[...].astype(bf16)
            l2 = load_lhs(lhs, (i + 1) % N_DEV, lhs_sem)
            l2.wait()
            mm(lhs, pan_ccw, acc_ccw)
            send_ccw[...] = acc_ccw[...].astype(bf16)

            for t in range(N_DEV - 1):
                if k > 0:
                    pl.semaphore_wait(credit_cw, 1)
                    pl.semaphore_wait(credit_ccw, 1)
                rdma_cw = pltpu.make_async_remote_copy(
                    src_ref=send_cw, dst_ref=recv_cw,
                    send_sem=send_sem_cw, recv_sem=recv_sem_cw,
                    device_id=(right,), device_id_type=pl.DeviceIdType.MESH,
                )
                rdma_ccw = pltpu.make_async_remote_copy(
                    src_ref=send_ccw, dst_ref=recv_ccw,
                    send_sem=send_sem_ccw, recv_sem=recv_sem_ccw,
                    device_id=(left,), device_id_type=pl.DeviceIdType.MESH,
                )
                rdma_cw.start()
                rdma_ccw.start()

                c_cw = (i + 2 * N_DEV - 2 - t) % N_DEV
                c_ccw = (i + 2 + t) % N_DEV
                if t == 1:
                    l1 = load_lhs(lhs, c_cw, lhs_sem)
                    l1.wait()
                    mm(lhs, pan_cw, acc_cw)
                    l2 = load_lhs(lhs, c_ccw, lhs_sem)
                    l2.wait()
                    mm(lhs, pan_ccw, acc_ccw)
                else:
                    l1 = load_lhs(lhs, c_cw, lhs_sem)
                    l1.wait()
                    mm(lhs, pan_cw, acc_cw)
                    mm(lhs, pan_ccw, acc_ccw)

                last = t == N_DEV - 2
                rdma_cw.wait_recv()
                rdma_cw.wait_send()
                if not last:
                    send_cw[...] = (
                        acc_cw[...] + recv_cw[...].astype(f32)
                    ).astype(bf16)
                else:
                    acc_cw[...] = acc_cw[...] + recv_cw[...].astype(f32)
                if k < N_STEP - 1:
                    pl.semaphore_signal(
                        credit_cw, inc=1,
                        device_id=(left,), device_id_type=pl.DeviceIdType.MESH,
                    )
                if last:
                    ocp = pltpu.make_async_copy(
                        acc_cw, o_ref.at[:, :, pl.ds(col_cw, W)], out_sem)
                    ocp.start()
                    ocp.wait()

                rdma_ccw.wait_recv()
                rdma_ccw.wait_send()
                if not last:
                    send_ccw[...] = (
                        acc_ccw[...] + recv_ccw[...].astype(f32)
                    ).astype(bf16)
                else:
                    acc_ccw[...] = acc_ccw[...] + recv_ccw[...].astype(f32)
                if k < N_STEP - 1:
                    pl.semaphore_signal(
                        credit_ccw, inc=1,
                        device_id=(right,), device_id_type=pl.DeviceIdType.MESH,
                    )
                if last:
                    ocp = pltpu.make_async_copy(
                        acc_ccw, o_ref.at[:, :, pl.ds(col_ccw, W)], out_sem)
                    ocp.start()
                    ocp.wait()
                k += 1

    return pl.pallas_call(
        body,
        out_shape=jax.ShapeDtypeStruct((B, S_out, N), f32),
        in_specs=[
            pl.BlockSpec(memory_space=pl.ANY),
            pl.BlockSpec(memory_space=pl.ANY),
        ],
        out_specs=pl.BlockSpec(memory_space=pl.ANY),
        scratch_shapes=[
            pltpu.VMEM((B, S_out, K), bf16),
            pltpu.VMEM((K, W), bf16),
            pltpu.VMEM((K, W), bf16),
            pltpu.VMEM((B, S_out, W), f32),
            pltpu.VMEM((B, S_out, W), f32),
            pltpu.VMEM((B, S_out, W), bf16),
            pltpu.VMEM((B, S_out, W), bf16),
            pltpu.VMEM((B, S_out, W), bf16),
            pltpu.VMEM((B, S_out, W), bf16),
            pltpu.SemaphoreType.DMA,
            pltpu.SemaphoreType.DMA,
            pltpu.SemaphoreType.DMA,
            pltpu.SemaphoreType.DMA,
            pltpu.SemaphoreType.DMA,
            pltpu.SemaphoreType.DMA,
            pltpu.SemaphoreType.DMA,
            pltpu.SemaphoreType.REGULAR,
            pltpu.SemaphoreType.REGULAR,
        ],
        compiler_params=pltpu.CompilerParams(
            collective_id=0,
            vmem_limit_bytes=60 * 1024 * 1024,
        ),
    )(Ob, Wob)
